# Initial kernel scaffold; baseline (speedup 1.0000x reference)
import jax
import jax.numpy as jnp
from jax.experimental import pallas as pl


def kernel(point_cloud_features):
    raise NotImplementedError("write your pallas kernel here")


# speedup vs baseline: 6.0854x; 6.0854x over previous
import functools

import jax
import jax.numpy as jnp
import numpy as np
from jax import lax
from jax.experimental import pallas as pl
from jax.experimental.pallas import tpu as pltpu, tpu_sc as plsc

B = 4
N = 50000
F = 101
NVOX = 144
OVF = 144
MAXP = 20
NV = 160
FP = 112
TPB = 8
P = N // TPB
NGRP = (P + 15) // 16
CHUNK = 96
NIDX = CHUNK * 5
KCAP = 3072
ROWS32 = (B * N * F) // 32

_MESH = plsc.VectorSubcoreMesh(core_axis_name="c", subcore_axis_name="s")
_CP = pltpu.CompilerParams(use_tc_tiling_on_sc=False, needs_layout_passes=False)


def _body(pc_hbm, pc32_hbm, om_hbm, on_hbm,
          xyz, wbuf, hist, starts, scalev, npbuf, allh,
          kg, kvid, ix, rb, vbv, dest, acc, redbuf, outbuf,
          hists_sh, accs_sh, sem):
    c = lax.axis_index("c")
    s = lax.axis_index("s")
    b = c * 2 + s // TPB
    t = s % TPB
    g0 = s - t
    chunk_start = t * P
    flat_start = b * N + chunk_start

    iota = lax.iota(jnp.int32, 16)
    ones = jnp.ones((16,), jnp.int32)
    zeros16 = jnp.zeros((16,), jnp.int32)

    @pl.loop(0, NV // 16)
    def _zh(k):
        hist[pl.ds(k * 16, 16)] = zeros16

    @pl.loop(0, NV)
    def _za(v):
        for k in range(FP // 16):
            acc[v, pl.ds(k * 16, 16)] = jnp.zeros((16,), jnp.float32)

    pltpu.sync_copy(pc_hbm.at[b, pl.ds(chunk_start, P), pl.ds(0, 3)], xyz)

    @pl.loop(0, NGRP)
    def _p1(i):
        pvec = i * 16 + iota
        pv = jnp.minimum(pvec, P - 1)
        x = plsc.load_gather(xyz, [pv, zeros16])
        y = plsc.load_gather(xyz, [pv, ones])
        z = plsc.load_gather(xyz, [pv, ones + 1])
        valid = ((x >= -1.0) & (x < 3.0) & (y >= -1.0) & (y < 5.0)
                 & (z >= -1.0) & (z < 5.0) & (pvec < P))
        vcx = (x + 1.0).astype(jnp.int32)
        vcy = (y + 1.0).astype(jnp.int32)
        vcz = (z + 1.0).astype(jnp.int32)
        vid = jnp.where(valid, vcx * 36 + vcy * 6 + vcz, OVF)
        base = plsc.load_gather(hist, [vid])
        cnt, _ = plsc.scan_count(vid)
        plsc.addupdate_scatter(hist, [vid], ones)
        rank = jnp.minimum(base + cnt - 1, 31)
        wbuf[pl.ds(i * 16, 16)] = jnp.bitwise_or(
            vid, lax.shift_left(rank, 8))

    pltpu.sync_copy(hist, hists_sh.at[s])
    plsc.subcore_barrier()
    pltpu.sync_copy(hists_sh.at[pl.ds(g0, TPB)], allh)

    @pl.loop(0, NV // 16)
    def _p2(k):
        st = zeros16
        tot = zeros16
        for j in range(TPB):
            h = allh[j, pl.ds(k * 16, 16)]
            mask = lax.full((16,), (j < t).astype(jnp.int32), jnp.int32)
            st = st + h * mask
            tot = tot + h
        starts[pl.ds(k * 16, 16)] = st
        npv = jnp.minimum(tot, MAXP)
        npbuf[pl.ds(k * 16, 16)] = npv
        nf = jnp.maximum(npv, 1).astype(jnp.float32)
        scalev[pl.ds(k * 16, 16)] = float(MAXP) / (nf * nf)

    def _p2b(i, m):
        w = wbuf[pl.ds(i * 16, 16)]
        vid = jnp.bitwise_and(w, 255)
        rank = lax.shift_right_logical(w, 8)
        stv = plsc.load_gather(starts, [vid])
        keep = (vid != OVF) & (stv + rank < MAXP)
        gflat = flat_start + i * 16 + iota
        plsc.store_compressed(kg.at[pl.ds(m, 16)], gflat, mask=keep)
        plsc.store_compressed(kvid.at[pl.ds(m, 16)], vid, mask=keep)
        return m + jnp.sum(keep.astype(jnp.int32))

    m = lax.fori_loop(0, NGRP, _p2b, jnp.int32(0))

    @pl.loop(0, CHUNK // 16)
    def _pad(j):
        kg[pl.ds(m + j * 16, 16)] = lax.full((16,), flat_start, jnp.int32)
        kvid[pl.ds(m + j * 16, 16)] = lax.full((16,), NV - 1, jnp.int32)

    nrounds = (m + CHUNK - 1) // CHUNK

    def _round(r, _):
        base_off = r * CHUNK

        @pl.loop(0, CHUNK // 16)
        def _prep(grp):
            pos = base_off + grp * 16
            gval = kg[pl.ds(pos, 16)]
            a = gval * F
            c0 = lax.shift_right_logical(a, 5)
            off = jnp.bitwise_and(a, 31)
            il = grp * 16 + iota
            rb[pl.ds(grp * 16, 16)] = il * 160 + off
            vbv[pl.ds(grp * 16, 16)] = kvid[pl.ds(pos, 16)]
            i5 = il * 5
            for kk in range(5):
                plsc.store_scatter(ix, [i5 + kk], c0 + kk)

        pltpu.async_copy(pc32_hbm.at[ix], dest, sem).wait()

        @pl.loop(0, CHUNK // 16)
        def _accum(grp):
            vb = vbv[pl.ds(grp * 16, 16)]

            def fj(j, b2):
                val = plsc.load_gather(
                    dest, [lax.shift_right_logical(b2, 5),
                           jnp.bitwise_and(b2, 31)])
                jv = lax.full((16,), j, jnp.int32)
                plsc.addupdate_scatter(acc, [vb, jv], val)
                return b2 + 1

            lax.fori_loop(0, F, fj, rb[pl.ds(grp * 16, 16)])

        return 0

    lax.fori_loop(0, nrounds, _round, 0)

    pltpu.sync_copy(acc, accs_sh.at[s])
    plsc.subcore_barrier()
    row0 = t * (NV // TPB)

    @pl.loop(0, NV // TPB)
    def _zo(v):
        for k in range(FP // 16):
            outbuf[v, pl.ds(k * 16, 16)] = jnp.zeros((16,), jnp.float32)

    @pl.loop(0, TPB)
    def _red(j):
        pltpu.sync_copy(accs_sh.at[g0 + j, pl.ds(row0, NV // TPB)], redbuf)

        @pl.loop(0, NV // TPB)
        def _addrows(v):
            for k in range(FP // 16):
                sl = pl.ds(k * 16, 16)
                outbuf[v, sl] = outbuf[v, sl] + redbuf[v, sl]

    @pl.loop(0, NV // TPB)
    def _scale(v):
        sc = plsc.load_gather(scalev, [lax.full((16,), row0 + v, jnp.int32)])
        for k in range(FP // 16):
            sl = pl.ds(k * 16, 16)
            outbuf[v, sl] = outbuf[v, sl] * sc

    pltpu.sync_copy(outbuf, om_hbm.at[b, pl.ds(row0, NV // TPB)])

    @pl.when(t == 0)
    def _np():
        pltpu.sync_copy(npbuf, on_hbm.at[b])


_voxel_kernel = functools.partial(
    pl.kernel, _body, mesh=_MESH, compiler_params=_CP,
    out_type=(jax.ShapeDtypeStruct((B, NV, FP), jnp.float32),
              jax.ShapeDtypeStruct((B, NV), jnp.int32)),
    scratch_types=[
        pltpu.VMEM((P, 3), jnp.float32),
        pltpu.VMEM((NGRP * 16,), jnp.int32),
        pltpu.VMEM((NV,), jnp.int32),
        pltpu.VMEM((NV,), jnp.int32),
        pltpu.VMEM((NV,), jnp.float32),
        pltpu.VMEM((NV,), jnp.int32),
        pltpu.VMEM((TPB, NV), jnp.int32),
        pltpu.VMEM((KCAP,), jnp.int32),
        pltpu.VMEM((KCAP,), jnp.int32),
        pltpu.VMEM((NIDX,), jnp.int32),
        pltpu.VMEM((CHUNK,), jnp.int32),
        pltpu.VMEM((CHUNK,), jnp.int32),
        pltpu.VMEM((NIDX, 32), jnp.float32),
        pltpu.VMEM((NV, FP), jnp.float32),
        pltpu.VMEM((NV // TPB, FP), jnp.float32),
        pltpu.VMEM((NV // TPB, FP), jnp.float32),
        pltpu.VMEM_SHARED((16, NV), jnp.int32),
        pltpu.VMEM_SHARED((16, NV, FP), jnp.float32),
        pltpu.SemaphoreType.DMA,
    ],
)()

_lin = np.arange(NVOX)
_COORDS = np.stack([_lin % 6, (_lin // 6) % 6, _lin // 36], axis=1).astype(np.int32)


def kernel(point_cloud_features):
    pc = point_cloud_features
    pc32 = pc.reshape(ROWS32, 32)
    out_mean, out_np = _voxel_kernel(pc, pc32)
    points_mean = out_mean[:, :NVOX, :F]
    num_points = out_np[:, :NVOX]
    coords = jnp.broadcast_to(jnp.asarray(_COORDS), (B, NVOX, 3))
    return points_mean, coords, num_points


# speedup vs baseline: 8.3596x; 1.3737x over previous
import functools

import jax
import jax.numpy as jnp
import numpy as np
from jax import lax
from jax.experimental import pallas as pl
from jax.experimental.pallas import tpu as pltpu, tpu_sc as plsc

B = 4
N = 50000
F = 101
NVOX = 144
OVF = 144
MAXP = 20
NV = 160
FP = 112
TPB = 8
P = N // TPB
NGRP = (P + 15) // 16
CHUNK = 96
NIDX = CHUNK * 5
KCAP = 3072
ROWS32 = (B * N * F) // 32

_MESH = plsc.VectorSubcoreMesh(core_axis_name="c", subcore_axis_name="s")
_CP = pltpu.CompilerParams(use_tc_tiling_on_sc=False, needs_layout_passes=False)


def _body(xyzt_hbm, pc32_hbm, om_hbm, on_hbm,
          xb, yb, zb, wbuf, hist, starts, scalev, npbuf, allh,
          kg, kvid, ix, rb, vbv, dest, acc, redbuf, outbuf,
          hists_sh, accs_sh, sem):
    c = lax.axis_index("c")
    s = lax.axis_index("s")
    b = c * 2 + s // TPB
    t = s % TPB
    g0 = s - t
    chunk_start = t * P
    flat_start = b * N + chunk_start

    iota = lax.iota(jnp.int32, 16)
    ones = jnp.ones((16,), jnp.int32)
    zeros16 = jnp.zeros((16,), jnp.int32)

    @pl.loop(0, NV // 16)
    def _zh(k):
        hist[pl.ds(k * 16, 16)] = zeros16

    @pl.loop(0, NV)
    def _za(v):
        for k in range(FP // 16):
            acc[v, pl.ds(k * 16, 16)] = jnp.zeros((16,), jnp.float32)

    st_al = (chunk_start // 8) * 8
    delta = chunk_start - st_al
    pltpu.sync_copy(xyzt_hbm.at[b, 0, pl.ds(st_al, P + 14)], xb)
    pltpu.sync_copy(xyzt_hbm.at[b, 1, pl.ds(st_al, P + 14)], yb)
    pltpu.sync_copy(xyzt_hbm.at[b, 2, pl.ds(st_al, P + 14)], zb)

    @pl.loop(0, NGRP)
    def _p1(i):
        pvec = i * 16 + iota
        x = xb[pl.ds(i * 16 + delta, 16)]
        y = yb[pl.ds(i * 16 + delta, 16)]
        z = zb[pl.ds(i * 16 + delta, 16)]
        valid = ((x >= -1.0) & (x < 3.0) & (y >= -1.0) & (y < 5.0)
                 & (z >= -1.0) & (z < 5.0) & (pvec < P))
        vcx = (x + 1.0).astype(jnp.int32)
        vcy = (y + 1.0).astype(jnp.int32)
        vcz = (z + 1.0).astype(jnp.int32)
        vid = jnp.where(valid, vcx * 36 + vcy * 6 + vcz, OVF)
        base = plsc.load_gather(hist, [vid])
        cnt, _ = plsc.scan_count(vid)
        plsc.addupdate_scatter(hist, [vid], ones)
        rank = jnp.minimum(base + cnt - 1, 31)
        wbuf[pl.ds(i * 16, 16)] = jnp.bitwise_or(
            vid, lax.shift_left(rank, 8))

    pltpu.sync_copy(hist, hists_sh.at[s])
    plsc.subcore_barrier()
    pltpu.sync_copy(hists_sh.at[pl.ds(g0, TPB)], allh)

    @pl.loop(0, NV // 16)
    def _p2(k):
        st = zeros16
        tot = zeros16
        for j in range(TPB):
            h = allh[j, pl.ds(k * 16, 16)]
            mask = lax.full((16,), (j < t).astype(jnp.int32), jnp.int32)
            st = st + h * mask
            tot = tot + h
        starts[pl.ds(k * 16, 16)] = st
        npv = jnp.minimum(tot, MAXP)
        npbuf[pl.ds(k * 16, 16)] = npv
        nf = jnp.maximum(npv, 1).astype(jnp.float32)
        scalev[pl.ds(k * 16, 16)] = float(MAXP) / (nf * nf)

    def _p2b(i, m):
        w = wbuf[pl.ds(i * 16, 16)]
        vid = jnp.bitwise_and(w, 255)
        rank = lax.shift_right_logical(w, 8)
        stv = plsc.load_gather(starts, [vid])
        keep = (vid != OVF) & (stv + rank < MAXP)
        gflat = flat_start + i * 16 + iota
        plsc.store_compressed(kg.at[pl.ds(m, 16)], gflat, mask=keep)
        plsc.store_compressed(kvid.at[pl.ds(m, 16)], vid, mask=keep)
        return m + jnp.sum(keep.astype(jnp.int32))

    m = lax.fori_loop(0, NGRP, _p2b, jnp.int32(0))

    @pl.loop(0, CHUNK // 16)
    def _pad(j):
        kg[pl.ds(m + j * 16, 16)] = lax.full((16,), flat_start, jnp.int32)
        kvid[pl.ds(m + j * 16, 16)] = lax.full((16,), NV - 1, jnp.int32)

    nrounds = (m + CHUNK - 1) // CHUNK

    def _round(r, _):
        base_off = r * CHUNK

        @pl.loop(0, CHUNK // 16)
        def _prep(grp):
            pos = base_off + grp * 16
            gval = kg[pl.ds(pos, 16)]
            a = gval * F
            c0 = lax.shift_right_logical(a, 5)
            off = jnp.bitwise_and(a, 31)
            il = grp * 16 + iota
            rb[pl.ds(grp * 16, 16)] = il * 160 + off
            vbv[pl.ds(grp * 16, 16)] = kvid[pl.ds(pos, 16)]
            i5 = il * 5
            for kk in range(5):
                plsc.store_scatter(ix, [i5 + kk], c0 + kk)

        pltpu.async_copy(pc32_hbm.at[ix], dest, sem).wait()

        @pl.loop(0, CHUNK // 16)
        def _accum(grp):
            vb = vbv[pl.ds(grp * 16, 16)]

            def fj(j, b2):
                val = plsc.load_gather(
                    dest, [lax.shift_right_logical(b2, 5),
                           jnp.bitwise_and(b2, 31)])
                jv = lax.full((16,), j, jnp.int32)
                plsc.addupdate_scatter(acc, [vb, jv], val)
                return b2 + 1

            lax.fori_loop(0, F, fj, rb[pl.ds(grp * 16, 16)])

        return 0

    lax.fori_loop(0, nrounds, _round, 0)

    pltpu.sync_copy(acc, accs_sh.at[s])
    plsc.subcore_barrier()
    row0 = t * (NV // TPB)

    @pl.loop(0, NV // TPB)
    def _zo(v):
        for k in range(FP // 16):
            outbuf[v, pl.ds(k * 16, 16)] = jnp.zeros((16,), jnp.float32)

    @pl.loop(0, TPB)
    def _red(j):
        pltpu.sync_copy(accs_sh.at[g0 + j, pl.ds(row0, NV // TPB)], redbuf)

        @pl.loop(0, NV // TPB)
        def _addrows(v):
            for k in range(FP // 16):
                sl = pl.ds(k * 16, 16)
                outbuf[v, sl] = outbuf[v, sl] + redbuf[v, sl]

    @pl.loop(0, NV // TPB)
    def _scale(v):
        sc = plsc.load_gather(scalev, [lax.full((16,), row0 + v, jnp.int32)])
        for k in range(FP // 16):
            sl = pl.ds(k * 16, 16)
            outbuf[v, sl] = outbuf[v, sl] * sc

    pltpu.sync_copy(outbuf, om_hbm.at[b, pl.ds(row0, NV // TPB)])

    @pl.when(t == 0)
    def _np():
        pltpu.sync_copy(npbuf, on_hbm.at[b])


_voxel_kernel = functools.partial(
    pl.kernel, _body, mesh=_MESH, compiler_params=_CP,
    out_type=(jax.ShapeDtypeStruct((B, NV, FP), jnp.float32),
              jax.ShapeDtypeStruct((B, NV), jnp.int32)),
    scratch_types=[
        pltpu.VMEM((P + 14,), jnp.float32),
        pltpu.VMEM((P + 14,), jnp.float32),
        pltpu.VMEM((P + 14,), jnp.float32),
        pltpu.VMEM((NGRP * 16,), jnp.int32),
        pltpu.VMEM((NV,), jnp.int32),
        pltpu.VMEM((NV,), jnp.int32),
        pltpu.VMEM((NV,), jnp.float32),
        pltpu.VMEM((NV,), jnp.int32),
        pltpu.VMEM((TPB, NV), jnp.int32),
        pltpu.VMEM((KCAP,), jnp.int32),
        pltpu.VMEM((KCAP,), jnp.int32),
        pltpu.VMEM((NIDX,), jnp.int32),
        pltpu.VMEM((CHUNK,), jnp.int32),
        pltpu.VMEM((CHUNK,), jnp.int32),
        pltpu.VMEM((NIDX, 32), jnp.float32),
        pltpu.VMEM((NV, FP), jnp.float32),
        pltpu.VMEM((NV // TPB, FP), jnp.float32),
        pltpu.VMEM((NV // TPB, FP), jnp.float32),
        pltpu.VMEM_SHARED((16, NV), jnp.int32),
        pltpu.VMEM_SHARED((16, NV, FP), jnp.float32),
        pltpu.SemaphoreType.DMA,
    ],
)()

_lin = np.arange(NVOX)
_COORDS = np.stack([_lin % 6, (_lin // 6) % 6, _lin // 36], axis=1).astype(np.int32)


def kernel(point_cloud_features):
    pc = point_cloud_features
    xyzt = jnp.pad(jnp.transpose(pc[:, :, :3], (0, 2, 1)),
                   ((0, 0), (0, 0), (0, 16)))
    pc32 = pc.reshape(ROWS32, 32)
    out_mean, out_np = _voxel_kernel(xyzt, pc32)
    points_mean = out_mean[:, :NVOX, :F]
    num_points = out_np[:, :NVOX]
    coords = jnp.broadcast_to(jnp.asarray(_COORDS), (B, NVOX, 3))
    return points_mean, coords, num_points


# speedup vs baseline: 17.4530x; 2.0878x over previous
import functools

import jax
import jax.numpy as jnp
import numpy as np
from jax import lax
from jax.experimental import pallas as pl
from jax.experimental.pallas import tpu as pltpu, tpu_sc as plsc

B = 4
N = 50000
F = 101
NVOX = 144
OVF = 144
MAXP = 20
NV = 160
FP = 112
TPB = 8
P = N // TPB
NGRP = (P + 15) // 16
GT = 391
NT = 50
KCAP = 3072

_MESH = plsc.VectorSubcoreMesh(core_axis_name="c", subcore_axis_name="s")
_CP = pltpu.CompilerParams(use_tc_tiling_on_sc=False, needs_layout_passes=False)


def _body(pcv_hbm, om_hbm, on_hbm,
          xb, yb, zb, pbufa, pbufb, wbuf, hist, starts, scalev, npbuf, allh,
          kg, kvid, pgt, pgl, acc, redbuf, outbuf,
          hists_sh, accs_sh, sema, semb):
    c = lax.axis_index("c")
    s = lax.axis_index("s")
    b = c * 2 + s // TPB
    t = s % TPB
    g0 = s - t
    chunk_start = t * P
    t0 = chunk_start // 128
    delta = chunk_start - t0 * 128

    iota = lax.iota(jnp.int32, 16)
    ones = jnp.ones((16,), jnp.int32)
    zeros16 = jnp.zeros((16,), jnp.int32)

    @pl.loop(0, NV // 16)
    def _zh(k):
        hist[pl.ds(k * 16, 16)] = zeros16

    @pl.loop(0, NV)
    def _za(v):
        for k in range(FP // 16):
            acc[v, pl.ds(k * 16, 16)] = jnp.zeros((16,), jnp.float32)

    pltpu.sync_copy(pcv_hbm.at[0, pl.ds(t0, NT), b, :], xb)
    pltpu.sync_copy(pcv_hbm.at[1, pl.ds(t0, NT), b, :], yb)
    pltpu.sync_copy(pcv_hbm.at[2, pl.ds(t0, NT), b, :], zb)

    @pl.loop(0, NGRP)
    def _p1(i):
        pvec = i * 16 + iota
        pw = pvec + delta
        pr = lax.shift_right_logical(pw, 7)
        pcid = jnp.bitwise_and(pw, 127)
        x = plsc.load_gather(xb, [pr, pcid])
        y = plsc.load_gather(yb, [pr, pcid])
        z = plsc.load_gather(zb, [pr, pcid])
        valid = ((x >= -1.0) & (x < 3.0) & (y >= -1.0) & (y < 5.0)
                 & (z >= -1.0) & (z < 5.0) & (pvec < P))
        vcx = (x + 1.0).astype(jnp.int32)
        vcy = (y + 1.0).astype(jnp.int32)
        vcz = (z + 1.0).astype(jnp.int32)
        vid = jnp.where(valid, vcx * 36 + vcy * 6 + vcz, OVF)
        base = plsc.load_gather(hist, [vid])
        cnt, _ = plsc.scan_count(vid)
        plsc.addupdate_scatter(hist, [vid], ones)
        rank = jnp.minimum(base + cnt - 1, 31)
        wbuf[pl.ds(i * 16, 16)] = jnp.bitwise_or(
            vid, lax.shift_left(rank, 8))

    pltpu.sync_copy(hist, hists_sh.at[s])
    plsc.subcore_barrier()
    pltpu.sync_copy(hists_sh.at[pl.ds(g0, TPB)], allh)

    @pl.loop(0, NV // 16)
    def _p2(k):
        st = zeros16
        tot = zeros16
        for j in range(TPB):
            h = allh[j, pl.ds(k * 16, 16)]
            mask = lax.full((16,), (j < t).astype(jnp.int32), jnp.int32)
            st = st + h * mask
            tot = tot + h
        starts[pl.ds(k * 16, 16)] = st
        npv = jnp.minimum(tot, MAXP)
        npbuf[pl.ds(k * 16, 16)] = npv
        nf = jnp.maximum(npv, 1).astype(jnp.float32)
        scalev[pl.ds(k * 16, 16)] = float(MAXP) / (nf * nf)

    def _p2b(i, m):
        w = wbuf[pl.ds(i * 16, 16)]
        vid = jnp.bitwise_and(w, 255)
        rank = lax.shift_right_logical(w, 8)
        stv = plsc.load_gather(starts, [vid])
        keep = (vid != OVF) & (stv + rank < MAXP)
        gloc = chunk_start + i * 16 + iota
        plsc.store_compressed(kg.at[pl.ds(m, 16)], gloc, mask=keep)
        plsc.store_compressed(kvid.at[pl.ds(m, 16)], vid, mask=keep)
        return m + jnp.sum(keep.astype(jnp.int32))

    m = lax.fori_loop(0, NGRP, _p2b, jnp.int32(0))

    kg[pl.ds(m, 16)] = lax.full((16,), chunk_start, jnp.int32)
    kvid[pl.ds(m, 16)] = lax.full((16,), NV - 1, jnp.int32)
    nkv = (m + 15) // 16

    def _prep(i, _):
        g = kg[pl.ds(i * 16, 16)]
        pgt[pl.ds(i * 16, 16)] = lax.shift_right_logical(g, 7) - t0
        pgl[pl.ds(i * 16, 16)] = jnp.bitwise_and(g, 127)
        return 0

    lax.fori_loop(0, nkv, _prep, 0)

    def _issue(f, buf, sem):
        pltpu.make_async_copy(
            pcv_hbm.at[f, pl.ds(t0, NT), b, :], buf, sem).start()

    def _wait(f, buf, sem):
        pltpu.make_async_copy(
            pcv_hbm.at[f, pl.ds(t0, NT), b, :], buf, sem).wait()

    def _consume(f, buf):
        fsplat = lax.full((16,), f, jnp.int32)

        def _acc1(i, _):
            sl = pl.ds(i * 16, 16)
            val = plsc.load_gather(buf, [pgt[sl], pgl[sl]])
            plsc.addupdate_scatter(acc, [kvid[sl], fsplat], val)
            return 0

        lax.fori_loop(0, nkv, _acc1, 0)

    _issue(0, pbufa, sema)

    @pl.loop(0, (F - 1) // 2)
    def _planes(q):
        fa = q * 2
        _wait(fa, pbufa, sema)
        _issue(fa + 1, pbufb, semb)
        _consume(fa, pbufa)
        _wait(fa + 1, pbufb, semb)
        _issue(fa + 2, pbufa, sema)
        _consume(fa + 1, pbufb)

    _wait(F - 1, pbufa, sema)
    _consume(F - 1, pbufa)

    pltpu.sync_copy(acc, accs_sh.at[s])
    plsc.subcore_barrier()
    row0 = t * (NV // TPB)

    @pl.loop(0, NV // TPB)
    def _zo(v):
        for k in range(FP // 16):
            outbuf[v, pl.ds(k * 16, 16)] = jnp.zeros((16,), jnp.float32)

    @pl.loop(0, TPB)
    def _red(j):
        pltpu.sync_copy(accs_sh.at[g0 + j, pl.ds(row0, NV // TPB)], redbuf)

        @pl.loop(0, NV // TPB)
        def _addrows(v):
            for k in range(FP // 16):
                sl = pl.ds(k * 16, 16)
                outbuf[v, sl] = outbuf[v, sl] + redbuf[v, sl]

    @pl.loop(0, NV // TPB)
    def _scale(v):
        sc = plsc.load_gather(scalev, [lax.full((16,), row0 + v, jnp.int32)])
        for k in range(FP // 16):
            sl = pl.ds(k * 16, 16)
            outbuf[v, sl] = outbuf[v, sl] * sc

    pltpu.sync_copy(outbuf, om_hbm.at[b, pl.ds(row0, NV // TPB)])

    @pl.when(t == 0)
    def _np():
        pltpu.sync_copy(npbuf, on_hbm.at[b])


_voxel_kernel = functools.partial(
    pl.kernel, _body, mesh=_MESH, compiler_params=_CP,
    out_type=(jax.ShapeDtypeStruct((B, NV, FP), jnp.float32),
              jax.ShapeDtypeStruct((B, NV), jnp.int32)),
    scratch_types=[
        pltpu.VMEM((NT, 128), jnp.float32),
        pltpu.VMEM((NT, 128), jnp.float32),
        pltpu.VMEM((NT, 128), jnp.float32),
        pltpu.VMEM((NT, 128), jnp.float32),
        pltpu.VMEM((NT, 128), jnp.float32),
        pltpu.VMEM((NGRP * 16,), jnp.int32),
        pltpu.VMEM((NV,), jnp.int32),
        pltpu.VMEM((NV,), jnp.int32),
        pltpu.VMEM((NV,), jnp.float32),
        pltpu.VMEM((NV,), jnp.int32),
        pltpu.VMEM((TPB, NV), jnp.int32),
        pltpu.VMEM((KCAP,), jnp.int32),
        pltpu.VMEM((KCAP,), jnp.int32),
        pltpu.VMEM((KCAP,), jnp.int32),
        pltpu.VMEM((KCAP,), jnp.int32),
        pltpu.VMEM((NV, FP), jnp.float32),
        pltpu.VMEM((NV // TPB, FP), jnp.float32),
        pltpu.VMEM((NV // TPB, FP), jnp.float32),
        pltpu.VMEM_SHARED((16, NV), jnp.int32),
        pltpu.VMEM_SHARED((16, NV, FP), jnp.float32),
        pltpu.SemaphoreType.DMA,
        pltpu.SemaphoreType.DMA,
    ],
)()

_lin = np.arange(NVOX)
_COORDS = np.stack([_lin % 6, (_lin // 6) % 6, _lin // 36], axis=1).astype(np.int32)


def kernel(point_cloud_features):
    pc = point_cloud_features
    pcp = jnp.pad(pc, ((0, 0), (0, GT * 128 - N), (0, 0)))
    pcv = jnp.transpose(pcp.reshape(B, GT, 128, F), (3, 1, 0, 2))
    out_mean, out_np = _voxel_kernel(pcv)
    points_mean = out_mean[:, :NVOX, :F]
    num_points = out_np[:, :NVOX]
    coords = jnp.broadcast_to(jnp.asarray(_COORDS), (B, NVOX, 3))
    return points_mean, coords, num_points


# speedup vs baseline: 18.0834x; 1.0361x over previous
import functools

import jax
import jax.numpy as jnp
import numpy as np
from jax import lax
from jax.experimental import pallas as pl
from jax.experimental.pallas import tpu as pltpu, tpu_sc as plsc

B = 4
N = 50000
F = 101
NVOX = 144
OVF = 144
SENT = 255
MAXP = 20
NV = 160
FP = 112
TPB = 8
P = N // TPB
NGRP = (P + 15) // 16
GT = 391
NT = 50
KCAP = 3072

_MESH = plsc.VectorSubcoreMesh(core_axis_name="c", subcore_axis_name="s")
_CP = pltpu.CompilerParams(use_tc_tiling_on_sc=False, needs_layout_passes=False)


def _select_body(xyzt_hbm, kg_hbm, kv_hbm, sc_hbm, on_hbm,
                 xb, yb, zb, wbuf, hist, starts, scalev, npbuf, allh,
                 kg, kvid, hists_sh):
    c = lax.axis_index("c")
    s = lax.axis_index("s")
    b = c * 2 + s // TPB
    t = s % TPB
    g0 = s - t
    chunk_start = t * P
    st_al = (chunk_start // 8) * 8
    delta = chunk_start - st_al

    iota = lax.iota(jnp.int32, 16)
    ones = jnp.ones((16,), jnp.int32)
    zeros16 = jnp.zeros((16,), jnp.int32)

    @pl.loop(0, NV // 16)
    def _zh(k):
        hist[pl.ds(k * 16, 16)] = zeros16

    @pl.loop(0, KCAP // 16)
    def _zk(k):
        kvid[pl.ds(k * 16, 16)] = lax.full((16,), SENT, jnp.int32)

    pltpu.sync_copy(xyzt_hbm.at[b, 0, pl.ds(st_al, P + 14)], xb)
    pltpu.sync_copy(xyzt_hbm.at[b, 1, pl.ds(st_al, P + 14)], yb)
    pltpu.sync_copy(xyzt_hbm.at[b, 2, pl.ds(st_al, P + 14)], zb)

    @pl.loop(0, NGRP)
    def _p1(i):
        pvec = i * 16 + iota
        x = xb[pl.ds(i * 16 + delta, 16)]
        y = yb[pl.ds(i * 16 + delta, 16)]
        z = zb[pl.ds(i * 16 + delta, 16)]
        valid = ((x >= -1.0) & (x < 3.0) & (y >= -1.0) & (y < 5.0)
                 & (z >= -1.0) & (z < 5.0) & (pvec < P))
        vcx = (x + 1.0).astype(jnp.int32)
        vcy = (y + 1.0).astype(jnp.int32)
        vcz = (z + 1.0).astype(jnp.int32)
        vid = jnp.where(valid, vcx * 36 + vcy * 6 + vcz, OVF)
        base = plsc.load_gather(hist, [vid])
        cnt, _ = plsc.scan_count(vid)
        plsc.addupdate_scatter(hist, [vid], ones)
        rank = jnp.minimum(base + cnt - 1, 31)
        wbuf[pl.ds(i * 16, 16)] = jnp.bitwise_or(
            vid, lax.shift_left(rank, 8))

    pltpu.sync_copy(hist, hists_sh.at[s])
    plsc.subcore_barrier()
    pltpu.sync_copy(hists_sh.at[pl.ds(g0, TPB)], allh)

    @pl.loop(0, NV // 16)
    def _p2(k):
        st = zeros16
        tot = zeros16
        for j in range(TPB):
            h = allh[j, pl.ds(k * 16, 16)]
            mask = lax.full((16,), (j < t).astype(jnp.int32), jnp.int32)
            st = st + h * mask
            tot = tot + h
        starts[pl.ds(k * 16, 16)] = st
        npv = jnp.minimum(tot, MAXP)
        npbuf[pl.ds(k * 16, 16)] = npv
        nf = jnp.maximum(npv, 1).astype(jnp.float32)
        scalev[pl.ds(k * 16, 16)] = float(MAXP) / (nf * nf)

    def _p2b(i, m):
        w = wbuf[pl.ds(i * 16, 16)]
        vid = jnp.bitwise_and(w, 255)
        rank = lax.shift_right_logical(w, 8)
        stv = plsc.load_gather(starts, [vid])
        keep = (vid != OVF) & (stv + rank < MAXP)
        gloc = chunk_start + i * 16 + iota
        plsc.store_compressed(kg.at[pl.ds(m, 16)], gloc, mask=keep)
        plsc.store_compressed(kvid.at[pl.ds(m, 16)], vid, mask=keep)
        return m + jnp.sum(keep.astype(jnp.int32))

    m = lax.fori_loop(0, NGRP, _p2b, jnp.int32(0))

    kg[pl.ds(m, 16)] = lax.full((16,), chunk_start, jnp.int32)
    kvid[pl.ds(m, 16)] = jnp.where(
        iota < (((m + 15) // 16) * 16 - m),
        lax.full((16,), NV - 1, jnp.int32), lax.full((16,), SENT, jnp.int32))

    pltpu.sync_copy(kg, kg_hbm.at[b, t])
    pltpu.sync_copy(kvid, kv_hbm.at[b, t])

    @pl.when(t == 0)
    def _aux():
        pltpu.sync_copy(npbuf, on_hbm.at[b])
        pltpu.sync_copy(scalev, sc_hbm.at[b])


def _accum_body(pcv_hbm, kg_hbm, kv_hbm, sc_hbm, om_hbm,
                pbufa, pbufb, kg, kvid, pgt, pgl, scalev,
                acc, redbuf, outbuf, accs_sh, sema, semb):
    c = lax.axis_index("c")
    s = lax.axis_index("s")
    b = c * 2 + s // TPB
    t = s % TPB
    g0 = s - t
    chunk_start = t * P
    t0 = chunk_start // 128

    iota = lax.iota(jnp.int32, 16)

    @pl.loop(0, NV)
    def _za(v):
        for k in range(FP // 16):
            acc[v, pl.ds(k * 16, 16)] = jnp.zeros((16,), jnp.float32)

    pltpu.sync_copy(kg_hbm.at[b, t], kg)
    pltpu.sync_copy(kv_hbm.at[b, t], kvid)
    pltpu.sync_copy(sc_hbm.at[b], scalev)

    def _scan_cond(carry):
        i, go = carry
        return go

    def _scan_step(carry):
        i, _ = carry
        v = kvid[pl.ds(i * 16, 16)]
        nsent = jnp.sum((v == SENT).astype(jnp.int32))
        return (i + 1, (nsent == 0) & (i + 1 < KCAP // 16))

    nkv0 = lax.while_loop(_scan_cond, _scan_step, (jnp.int32(0), True))[0]
    v0 = kvid[pl.ds((nkv0 - 1) * 16, 16)]
    nkv = nkv0 - (jnp.sum((v0 == SENT).astype(jnp.int32)) > 0).astype(jnp.int32)

    def _prep(i, _):
        g = kg[pl.ds(i * 16, 16)]
        pgt[pl.ds(i * 16, 16)] = lax.shift_right_logical(g, 7) - t0
        pgl[pl.ds(i * 16, 16)] = jnp.bitwise_and(g, 127)
        return 0

    lax.fori_loop(0, nkv, _prep, 0)

    def _issue(f, buf, sem):
        pltpu.make_async_copy(
            pcv_hbm.at[f, pl.ds(t0, NT), b, :], buf, sem).start()

    def _wait(f, buf, sem):
        pltpu.make_async_copy(
            pcv_hbm.at[f, pl.ds(t0, NT), b, :], buf, sem).wait()

    def _consume(f, buf):
        fsplat = lax.full((16,), f, jnp.int32)

        def _acc1(i, _):
            sl = pl.ds(i * 16, 16)
            val = plsc.load_gather(buf, [pgt[sl], pgl[sl]])
            plsc.addupdate_scatter(acc, [kvid[sl], fsplat], val)
            return 0

        lax.fori_loop(0, nkv, _acc1, 0)

    _issue(0, pbufa, sema)

    @pl.loop(0, (F - 1) // 2)
    def _planes(q):
        fa = q * 2
        _wait(fa, pbufa, sema)
        _issue(fa + 1, pbufb, semb)
        _consume(fa, pbufa)
        _wait(fa + 1, pbufb, semb)
        _issue(fa + 2, pbufa, sema)
        _consume(fa + 1, pbufb)

    _wait(F - 1, pbufa, sema)
    _consume(F - 1, pbufa)

    pltpu.sync_copy(acc, accs_sh.at[s])
    plsc.subcore_barrier()
    row0 = t * (NV // TPB)

    @pl.loop(0, NV // TPB)
    def _zo(v):
        for k in range(FP // 16):
            outbuf[v, pl.ds(k * 16, 16)] = jnp.zeros((16,), jnp.float32)

    @pl.loop(0, TPB)
    def _red(j):
        pltpu.sync_copy(accs_sh.at[g0 + j, pl.ds(row0, NV // TPB)], redbuf)

        @pl.loop(0, NV // TPB)
        def _addrows(v):
            for k in range(FP // 16):
                sl = pl.ds(k * 16, 16)
                outbuf[v, sl] = outbuf[v, sl] + redbuf[v, sl]

    @pl.loop(0, NV // TPB)
    def _scale(v):
        sc = plsc.load_gather(scalev, [lax.full((16,), row0 + v, jnp.int32)])
        for k in range(FP // 16):
            sl = pl.ds(k * 16, 16)
            outbuf[v, sl] = outbuf[v, sl] * sc

    pltpu.sync_copy(outbuf, om_hbm.at[b, pl.ds(row0, NV // TPB)])


_select_kernel = functools.partial(
    pl.kernel, _select_body, mesh=_MESH, compiler_params=_CP,
    out_type=(jax.ShapeDtypeStruct((B, TPB, KCAP), jnp.int32),
              jax.ShapeDtypeStruct((B, TPB, KCAP), jnp.int32),
              jax.ShapeDtypeStruct((B, NV), jnp.float32),
              jax.ShapeDtypeStruct((B, NV), jnp.int32)),
    scratch_types=[
        pltpu.VMEM((P + 14,), jnp.float32),
        pltpu.VMEM((P + 14,), jnp.float32),
        pltpu.VMEM((P + 14,), jnp.float32),
        pltpu.VMEM((NGRP * 16,), jnp.int32),
        pltpu.VMEM((NV,), jnp.int32),
        pltpu.VMEM((NV,), jnp.int32),
        pltpu.VMEM((NV,), jnp.float32),
        pltpu.VMEM((NV,), jnp.int32),
        pltpu.VMEM((TPB, NV), jnp.int32),
        pltpu.VMEM((KCAP,), jnp.int32),
        pltpu.VMEM((KCAP,), jnp.int32),
        pltpu.VMEM_SHARED((16, NV), jnp.int32),
    ],
)()

_accum_kernel = functools.partial(
    pl.kernel, _accum_body, mesh=_MESH, compiler_params=_CP,
    out_type=jax.ShapeDtypeStruct((B, NV, FP), jnp.float32),
    scratch_types=[
        pltpu.VMEM((NT, 128), jnp.float32),
        pltpu.VMEM((NT, 128), jnp.float32),
        pltpu.VMEM((KCAP,), jnp.int32),
        pltpu.VMEM((KCAP,), jnp.int32),
        pltpu.VMEM((KCAP,), jnp.int32),
        pltpu.VMEM((KCAP,), jnp.int32),
        pltpu.VMEM((NV,), jnp.float32),
        pltpu.VMEM((NV, FP), jnp.float32),
        pltpu.VMEM((NV // TPB, FP), jnp.float32),
        pltpu.VMEM((NV // TPB, FP), jnp.float32),
        pltpu.VMEM_SHARED((16, NV, FP), jnp.float32),
        pltpu.SemaphoreType.DMA,
        pltpu.SemaphoreType.DMA,
    ],
)()

_lin = np.arange(NVOX)
_COORDS = np.stack([_lin % 6, (_lin // 6) % 6, _lin // 36], axis=1).astype(np.int32)


def kernel(point_cloud_features):
    pc = point_cloud_features
    xyzt = jnp.pad(jnp.transpose(pc[:, :, :3], (0, 2, 1)),
                   ((0, 0), (0, 0), (0, 16)))
    pcp = jnp.pad(pc, ((0, 0), (0, GT * 128 - N), (0, 0)))
    pcv = jnp.transpose(pcp.reshape(B, GT, 128, F), (3, 1, 0, 2))
    kg, kv, sc, out_np = _select_kernel(xyzt)
    out_mean = _accum_kernel(pcv, kg, kv, sc)
    points_mean = out_mean[:, :NVOX, :F]
    num_points = out_np[:, :NVOX]
    coords = jnp.broadcast_to(jnp.asarray(_COORDS), (B, NVOX, 3))
    return points_mean, coords, num_points
